# baseline (speedup 1.0000x reference)
import jax
import jax.numpy as jnp
from jax import lax
from jax.experimental import pallas as pl
from jax.experimental.pallas import tpu as pltpu
from jax.experimental.pallas import tpu_sc as plsc

D_MODEL = 128
MAX_REL = 128
SEQ_LEN = 1024
TBL = 2 * MAX_REL + 1
PAD = SEQ_LEN - MAX_REL - 1
E_LEN = 2 * PAD + TBL

NUM_CORES = 2
NUM_SUBCORES = 16
NUM_WORKERS = NUM_CORES * NUM_SUBCORES
ROWS_PER_SUBCORE = SEQ_LEN // NUM_WORKERS
STRIDE = 8
BAND = STRIDE * ROWS_PER_SUBCORE
SPAN = STRIDE * (ROWS_PER_SUBCORE - 1)

M = 648
CHUNK = M + SPAN
TAIL = SEQ_LEN - M
PADW = 7
PAD_CHUNK = 128
LAG = 4
LANES = D_MODEL // 16


def _sc_body(table_hbm, out_hbm, e_sh, buf, sem_t, sem_m):
    c = lax.axis_index("c")
    s = lax.axis_index("s")

    @pl.when(s == 0)
    def _():
        pltpu.sync_copy(table_hbm, e_sh.at[pl.ds(PAD, TBL)])

    @pl.when(jnp.logical_and(s >= 1, s <= 2 * PADW))
    def _():
        w = s - 1
        side = w // PADW
        k = lax.min((w % PADW) * PAD_CHUNK, PAD - PAD_CHUNK)
        row = lax.select(side == 0, 0, TBL - 1)
        e_off = lax.select(side == 0, k, PAD + TBL + k)
        pltpu.sync_copy(table_hbm.at[pl.ds(row, 1)], buf.at[pl.ds(0, 1)])
        vals = [buf[0, pl.ds(lane * 16, 16)] for lane in range(LANES)]

        def fill(rr, carry):
            for lane in range(LANES):
                buf[rr, pl.ds(lane * 16, 16)] = vals[lane]
            return carry

        lax.fori_loop(1, PAD_CHUNK, fill, 0)
        pltpu.sync_copy(buf.at[pl.ds(0, PAD_CHUNK)],
                        e_sh.at[pl.ds(e_off, PAD_CHUNK)])

    plsc.subcore_barrier()

    u = c * NUM_SUBCORES + s
    i_first = BAND * (u // STRIDE) + (u % STRIDE)

    tails = []
    for t in range(ROWS_PER_SUBCORE):
        i = i_first + STRIDE * t
        tails.append(pltpu.async_copy(
            e_sh.at[pl.ds(SEQ_LEN - 1 - i + M, TAIL)],
            out_hbm.at[i, pl.ds(M, TAIL)], sem_t))

    a0 = SEQ_LEN - 1 - (i_first + SPAN)
    pltpu.sync_copy(e_sh.at[pl.ds(a0, CHUNK)], buf.at[pl.ds(0, CHUNK)])

    pending = []
    for t in range(ROWS_PER_SUBCORE):
        i = i_first + STRIDE * t
        pending.append(pltpu.async_copy(
            buf.at[pl.ds(SPAN - STRIDE * t, M)],
            out_hbm.at[i, pl.ds(0, M)], sem_m))
        if len(pending) > LAG:
            pending.pop(0).wait()
    for dsc in pending:
        dsc.wait()
    for dsc in tails:
        dsc.wait()


def kernel(seq_len, table):
    del seq_len
    mesh = plsc.VectorSubcoreMesh(
        core_axis_name="c",
        subcore_axis_name="s",
        num_cores=NUM_CORES,
        num_subcores=NUM_SUBCORES,
    )
    run = pl.kernel(
        _sc_body,
        out_type=jax.ShapeDtypeStruct((SEQ_LEN, SEQ_LEN, D_MODEL), jnp.float32),
        mesh=mesh,
        scratch_types=[
            pltpu.VMEM_SHARED((E_LEN, D_MODEL), jnp.float32),
            pltpu.VMEM((CHUNK, D_MODEL), jnp.float32),
            pltpu.SemaphoreType.DMA,
            pltpu.SemaphoreType.DMA,
        ],
    )
    return run(table)


# speedup vs baseline: 1.0113x; 1.0113x over previous
import jax
import jax.numpy as jnp
from jax import lax
from jax.experimental import pallas as pl
from jax.experimental.pallas import tpu as pltpu
from jax.experimental.pallas import tpu_sc as plsc

D_MODEL = 128
MAX_REL = 128
SEQ_LEN = 1024
TBL = 2 * MAX_REL + 1
PAD = SEQ_LEN - MAX_REL - 1
E_LEN = 2 * PAD + TBL

NUM_CORES = 2
NUM_SUBCORES = 16
NUM_WORKERS = NUM_CORES * NUM_SUBCORES
ROWS_PER_SUBCORE = SEQ_LEN // NUM_WORKERS
STRIDE = 8
BAND = STRIDE * ROWS_PER_SUBCORE
SPAN = STRIDE * (ROWS_PER_SUBCORE - 1)

M = 640
CHUNK = M + SPAN
TAIL = SEQ_LEN - M
PADW = 5
PAD_CHUNK = PAD // PADW
LAG = 4
LANES = D_MODEL // 16


def _sc_body(table_hbm, out_hbm, e_sh, buf, sem):
    c = lax.axis_index("c")
    s = lax.axis_index("s")

    @pl.when(s == 0)
    def _():
        pltpu.sync_copy(table_hbm, e_sh.at[pl.ds(PAD, TBL)])

    @pl.when(jnp.logical_and(s >= 1, s <= 2 * PADW))
    def _():
        w = s - 1
        side = w // PADW
        k = (w % PADW) * PAD_CHUNK
        row = lax.select(side == 0, 0, TBL - 1)
        e_off = lax.select(side == 0, k, PAD + TBL + k)
        pltpu.sync_copy(table_hbm.at[pl.ds(row, 1)], buf.at[pl.ds(0, 1)])
        vals = [buf[0, pl.ds(lane * 16, 16)] for lane in range(LANES)]

        def fill(rr, carry):
            for lane in range(LANES):
                buf[rr, pl.ds(lane * 16, 16)] = vals[lane]
            return carry

        lax.fori_loop(1, PAD_CHUNK, fill, 0)
        pltpu.sync_copy(buf.at[pl.ds(0, PAD_CHUNK)],
                        e_sh.at[pl.ds(e_off, PAD_CHUNK)])

    plsc.subcore_barrier()

    u = c * NUM_SUBCORES + s
    i_first = BAND * (u // STRIDE) + (u % STRIDE)
    a0 = SEQ_LEN - 1 - (i_first + SPAN)
    pltpu.sync_copy(e_sh.at[pl.ds(a0, CHUNK)], buf.at[pl.ds(0, CHUNK)])

    pending = []
    for t in range(ROWS_PER_SUBCORE):
        i = i_first + STRIDE * t
        row_descs = [
            pltpu.async_copy(
                buf.at[pl.ds(SPAN - STRIDE * t, M)],
                out_hbm.at[i, pl.ds(0, M)], sem),
            pltpu.async_copy(
                e_sh.at[pl.ds(SEQ_LEN - 1 - i + M, TAIL)],
                out_hbm.at[i, pl.ds(M, TAIL)], sem),
        ]
        pending.append(row_descs)
        if len(pending) > LAG:
            for dsc in pending.pop(0):
                dsc.wait()
    for row_descs in pending:
        for dsc in row_descs:
            dsc.wait()


def kernel(seq_len, table):
    del seq_len
    mesh = plsc.VectorSubcoreMesh(
        core_axis_name="c",
        subcore_axis_name="s",
        num_cores=NUM_CORES,
        num_subcores=NUM_SUBCORES,
    )
    run = pl.kernel(
        _sc_body,
        out_type=jax.ShapeDtypeStruct((SEQ_LEN, SEQ_LEN, D_MODEL), jnp.float32),
        mesh=mesh,
        scratch_types=[
            pltpu.VMEM_SHARED((E_LEN, D_MODEL), jnp.float32),
            pltpu.VMEM((CHUNK, D_MODEL), jnp.float32),
            pltpu.SemaphoreType.DMA,
        ],
    )
    return run(table)
